# baseline (speedup 1.0000x reference)
import functools

import jax
import jax.numpy as jnp
from jax.experimental import pallas as pl
from jax.experimental.pallas import tpu as pltpu

_H = 112
_NEG = -1e30
_PREC = jax.lax.Precision.HIGHEST


def _conv1_kernel(x_ref, w_ref, b_ref, o_ref):
    def body(o, _):
        acc = jnp.zeros((224, 224), jnp.float32) + b_ref[o]
        for p in range(3):
            for k in range(9):
                dy, dx = divmod(k, 3)
                acc += x_ref[p, dy:dy + 224, dx:dx + 224] * w_ref[o, p * 9 + k]
        o_ref[o] = jnp.maximum(acc, 0.0)
        return 0

    jax.lax.fori_loop(0, 64, body, 0, unroll=False)


def _dconv_kernel(x_ref, w_ref, b_ref, o_ref, *, rows, out_w, d):
    i = pl.program_id(0)
    r0 = i * rows
    w2 = x_ref.shape[1]
    cin = x_ref.shape[-1]
    cout = w_ref.shape[-1]
    o_ref[...] = jnp.zeros((rows, out_w, cout), jnp.float32) + b_ref[...][None]
    for k in range(9):
        dy, dx = divmod(k, 3)
        slab = x_ref[pl.ds(r0 + dy * d, rows), :, :]
        y = jnp.dot(slab.reshape(rows * w2, cin), w_ref[k],
                    preferred_element_type=jnp.float32,
                    precision=_PREC).reshape(rows, w2, cout)
        o_ref[...] += y[:, dx * d:dx * d + out_w, :]
    o_ref[...] = jnp.maximum(o_ref[...], 0.0)


def _gconv_kernel(x_ref, w_ref, b_ref, m_ref, o_ref, *, rows, d):
    i = pl.program_id(0)
    r0 = i * rows
    w2 = x_ref.shape[1]
    cin = x_ref.shape[-1]
    cout = w_ref.shape[-1]
    o_ref[...] = jnp.zeros((rows, _H, cout), jnp.float32) + b_ref[...][None]
    for k in range(9):
        dy, dx = divmod(k, 3)
        slab = x_ref[pl.ds(r0 + dy * d, rows), :, :]
        y = jnp.dot(slab.reshape(rows * w2, cin), w_ref[k],
                    preferred_element_type=jnp.float32,
                    precision=_PREC).reshape(rows, w2, cout)
        o_ref[...] += y[:, dx * d:dx * d + _H, :]
    o_ref[...] = jnp.maximum(o_ref[...], 0.0) * m_ref[...]


def _maxpool_kernel(x_ref, o_ref, *, rows):
    t0 = pl.program_id(0) * rows
    a00 = x_ref[pl.Slice(2 * t0, rows, 2), pl.Slice(0, _H, 2), :]
    a01 = x_ref[pl.Slice(2 * t0, rows, 2), pl.Slice(1, _H, 2), :]
    a10 = x_ref[pl.Slice(2 * t0 + 1, rows, 2), pl.Slice(0, _H, 2), :]
    a11 = x_ref[pl.Slice(2 * t0 + 1, rows, 2), pl.Slice(1, _H, 2), :]
    o_ref[...] = jnp.maximum(jnp.maximum(a00, a01), jnp.maximum(a10, a11))


def _pool_mask_kernel(x_ref, m_ref, o_ref, *, rows, d):
    t0 = pl.program_id(0) * rows
    best = jnp.full((rows, _H, x_ref.shape[-1]), _NEG, jnp.float32)
    for dy in (0, d):
        for dx in (0, d):
            v = m_ref[pl.ds(t0 + dy, rows), dx:dx + _H, :]
            c = x_ref[pl.ds(t0 + dy, rows), dx:dx + _H, :]
            best = jnp.maximum(best, jnp.where(v > 0, c, _NEG))
    best = jnp.where(best < -1e29, 0.0, best)
    o_ref[...] = best * m_ref[pl.ds(t0, rows), 0:_H, :]


def _full(shape):
    n = len(shape)
    return pl.BlockSpec(shape, lambda i: (0,) * n)


def _padded(x, d):
    return jnp.pad(x, ((d, d), (d, 8 - d), (0, 0)))


def _dconv(xp, w, b, *, rows, out_h, out_w, d):
    cout = w.shape[-1]
    return pl.pallas_call(
        functools.partial(_dconv_kernel, rows=rows, out_w=out_w, d=d),
        grid=(out_h // rows,),
        in_specs=[_full(xp.shape), _full(w.shape), _full(b.shape)],
        out_specs=pl.BlockSpec((rows, out_w, cout), lambda i: (i, 0, 0)),
        out_shape=jax.ShapeDtypeStruct((out_h, out_w, cout), jnp.float32),
    )(xp, w, b)


def _gconv(x, w, b, mask3, *, rows, d):
    cout = w.shape[-1]
    xp = _padded(x, d)
    return pl.pallas_call(
        functools.partial(_gconv_kernel, rows=rows, d=d),
        grid=(_H // rows,),
        in_specs=[_full(xp.shape), _full(w.shape), _full(b.shape),
                  pl.BlockSpec((rows, _H, 1), lambda i: (i, 0, 0))],
        out_specs=pl.BlockSpec((rows, _H, cout), lambda i: (i, 0, 0)),
        out_shape=jax.ShapeDtypeStruct((_H, _H, cout), jnp.float32),
    )(xp, w, b, mask3)


def _maxpool(x, *, rows):
    c = x.shape[-1]
    return pl.pallas_call(
        functools.partial(_maxpool_kernel, rows=rows),
        grid=(_H // rows,),
        in_specs=[_full(x.shape)],
        out_specs=pl.BlockSpec((rows, _H, c), lambda i: (i, 0, 0)),
        out_shape=jax.ShapeDtypeStruct((_H, _H, c), jnp.float32),
    )(x)


def _pool_mask(x, mask3, *, rows, d):
    c = x.shape[-1]
    xe = jnp.pad(x, ((0, 8), (0, 8), (0, 0)), mode="edge")
    me = jnp.pad(mask3, ((0, 8), (0, 8), (0, 0)), mode="edge")
    return pl.pallas_call(
        functools.partial(_pool_mask_kernel, rows=rows, d=d),
        grid=(_H // rows,),
        in_specs=[_full(xe.shape), _full(me.shape)],
        out_specs=pl.BlockSpec((rows, _H, c), lambda i: (i, 0, 0)),
        out_shape=jax.ShapeDtypeStruct((_H, _H, c), jnp.float32),
    )(xe, me)


def _taps(w):
    return jnp.transpose(w, (2, 3, 1, 0)).reshape(9, w.shape[1], w.shape[0])


def kernel(batch, pooling_mask, w1, b1, w2, b2, w3, b3, w4, b4, w5, b5,
           w6, b6, w7, b7, w8, b8, w9, b9, w10, b10):
    mask3 = pooling_mask[0].astype(jnp.float32)[:, :, None]

    x0p = jnp.pad(batch[0], ((0, 0), (1, 1), (1, 1)))
    x1pl = pl.pallas_call(
        _conv1_kernel,
        in_specs=[pl.BlockSpec(memory_space=pltpu.VMEM),
                  pl.BlockSpec(memory_space=pltpu.SMEM),
                  pl.BlockSpec(memory_space=pltpu.SMEM)],
        out_specs=pl.BlockSpec(memory_space=pltpu.VMEM),
        out_shape=jax.ShapeDtypeStruct((64, 224, 224), jnp.float32),
    )(x0p, w1.reshape(64, 27), b1)
    x1 = jnp.transpose(x1pl, (1, 2, 0))

    x2f = _dconv(_padded(x1, 1), _taps(w2), b2.reshape(1, -1), rows=16,
                 out_h=224, out_w=224, d=1)
    x2 = _maxpool(x2f, rows=16)
    x3 = _dconv(_padded(x2, 1), _taps(w3), b3.reshape(1, -1), rows=16,
                out_h=_H, out_w=_H, d=1)
    x4 = _dconv(_padded(x3, 1), _taps(w4), b4.reshape(1, -1), rows=16,
                out_h=_H, out_w=_H, d=1)

    n1 = _pool_mask(x4, mask3, rows=16, d=1)
    g5 = _gconv(n1, w5, b5.reshape(1, -1), mask3, rows=16, d=2)
    g6 = _gconv(g5, w6, b6.reshape(1, -1), mask3, rows=16, d=2)
    g7 = _gconv(g6, w7, b7.reshape(1, -1), mask3, rows=16, d=2)
    n2 = _pool_mask(g7, mask3, rows=16, d=2)
    g8 = _gconv(n2, w8, b8.reshape(1, -1), mask3, rows=16, d=4)
    g9 = _gconv(g8, w9, b9.reshape(1, -1), mask3, rows=16, d=4)
    g10 = _gconv(g9, w10, b10.reshape(1, -1), mask3, rows=16, d=4)

    return jnp.transpose(g10, (2, 0, 1))[None]


# speedup vs baseline: 3.2803x; 3.2803x over previous
import functools

import jax
import jax.numpy as jnp
from jax.experimental import pallas as pl
from jax.experimental.pallas import tpu as pltpu

_H = 112
_NEG = -1e30
_PREC = jax.lax.Precision.DEFAULT


def _conv1_kernel(x_ref, w_ref, b_ref, o_ref):
    def body(o, _):
        acc = jnp.zeros((224, 224), jnp.float32) + b_ref[o]
        for p in range(3):
            for k in range(9):
                dy, dx = divmod(k, 3)
                acc += x_ref[p, dy:dy + 224, dx:dx + 224] * w_ref[o, p * 9 + k]
        o_ref[o] = jnp.maximum(acc, 0.0)
        return 0

    jax.lax.fori_loop(0, 64, body, 0, unroll=False)


def _dconv_kernel(x_ref, w_ref, b_ref, o_ref, *, rows, out_w, d):
    i = pl.program_id(0)
    r0 = i * rows
    w2 = x_ref.shape[1]
    cin = x_ref.shape[-1]
    cout = w_ref.shape[-1]
    o_ref[...] = jnp.zeros((rows, out_w, cout), jnp.float32) + b_ref[...][None]
    for k in range(9):
        dy, dx = divmod(k, 3)
        slab = x_ref[pl.ds(r0 + dy * d, rows), :, :]
        y = jnp.dot(slab.reshape(rows * w2, cin), w_ref[k],
                    preferred_element_type=jnp.float32,
                    precision=_PREC).reshape(rows, w2, cout)
        o_ref[...] += y[:, dx * d:dx * d + out_w, :]
    o_ref[...] = jnp.maximum(o_ref[...], 0.0)


def _gconv_kernel(x_ref, w_ref, b_ref, m_ref, o_ref, *, rows, d):
    i = pl.program_id(0)
    r0 = i * rows
    w2 = x_ref.shape[1]
    cin = x_ref.shape[-1]
    cout = w_ref.shape[-1]
    o_ref[...] = jnp.zeros((rows, _H, cout), jnp.float32) + b_ref[...][None]
    for k in range(9):
        dy, dx = divmod(k, 3)
        slab = x_ref[pl.ds(r0 + dy * d, rows), :, :]
        y = jnp.dot(slab.reshape(rows * w2, cin), w_ref[k],
                    preferred_element_type=jnp.float32,
                    precision=_PREC).reshape(rows, w2, cout)
        o_ref[...] += y[:, dx * d:dx * d + _H, :]
    o_ref[...] = jnp.maximum(o_ref[...], 0.0) * m_ref[...]


def _maxpool_kernel(x_ref, o_ref, *, rows):
    t0 = pl.program_id(0) * rows
    a00 = x_ref[pl.Slice(2 * t0, rows, 2), pl.Slice(0, _H, 2), :]
    a01 = x_ref[pl.Slice(2 * t0, rows, 2), pl.Slice(1, _H, 2), :]
    a10 = x_ref[pl.Slice(2 * t0 + 1, rows, 2), pl.Slice(0, _H, 2), :]
    a11 = x_ref[pl.Slice(2 * t0 + 1, rows, 2), pl.Slice(1, _H, 2), :]
    o_ref[...] = jnp.maximum(jnp.maximum(a00, a01), jnp.maximum(a10, a11))


def _pool_mask_kernel(x_ref, m_ref, o_ref, *, rows, d):
    t0 = pl.program_id(0) * rows
    best = jnp.full((rows, _H, x_ref.shape[-1]), _NEG, jnp.float32)
    for dy in (0, d):
        for dx in (0, d):
            v = m_ref[pl.ds(t0 + dy, rows), dx:dx + _H, :]
            c = x_ref[pl.ds(t0 + dy, rows), dx:dx + _H, :]
            best = jnp.maximum(best, jnp.where(v > 0, c, _NEG))
    best = jnp.where(best < -1e29, 0.0, best)
    o_ref[...] = best * m_ref[pl.ds(t0, rows), 0:_H, :]


def _full(shape):
    n = len(shape)
    return pl.BlockSpec(shape, lambda i: (0,) * n)


def _padded(x, d):
    return jnp.pad(x, ((d, d), (d, 8 - d), (0, 0)))


def _dconv(xp, w, b, *, rows, out_h, out_w, d):
    cout = w.shape[-1]
    return pl.pallas_call(
        functools.partial(_dconv_kernel, rows=rows, out_w=out_w, d=d),
        grid=(out_h // rows,),
        in_specs=[_full(xp.shape), _full(w.shape), _full(b.shape)],
        out_specs=pl.BlockSpec((rows, out_w, cout), lambda i: (i, 0, 0)),
        out_shape=jax.ShapeDtypeStruct((out_h, out_w, cout), jnp.float32),
    )(xp, w, b)


def _gconv(x, w, b, mask3, *, rows, d):
    cout = w.shape[-1]
    xp = _padded(x, d)
    return pl.pallas_call(
        functools.partial(_gconv_kernel, rows=rows, d=d),
        grid=(_H // rows,),
        in_specs=[_full(xp.shape), _full(w.shape), _full(b.shape),
                  pl.BlockSpec((rows, _H, 1), lambda i: (i, 0, 0))],
        out_specs=pl.BlockSpec((rows, _H, cout), lambda i: (i, 0, 0)),
        out_shape=jax.ShapeDtypeStruct((_H, _H, cout), jnp.float32),
    )(xp, w, b, mask3)


def _maxpool(x, *, rows):
    c = x.shape[-1]
    return pl.pallas_call(
        functools.partial(_maxpool_kernel, rows=rows),
        grid=(_H // rows,),
        in_specs=[_full(x.shape)],
        out_specs=pl.BlockSpec((rows, _H, c), lambda i: (i, 0, 0)),
        out_shape=jax.ShapeDtypeStruct((_H, _H, c), jnp.float32),
    )(x)


def _pool_mask(x, mask3, *, rows, d):
    c = x.shape[-1]
    xe = jnp.pad(x, ((0, 8), (0, 8), (0, 0)), mode="edge")
    me = jnp.pad(mask3, ((0, 8), (0, 8), (0, 0)), mode="edge")
    return pl.pallas_call(
        functools.partial(_pool_mask_kernel, rows=rows, d=d),
        grid=(_H // rows,),
        in_specs=[_full(xe.shape), _full(me.shape)],
        out_specs=pl.BlockSpec((rows, _H, c), lambda i: (i, 0, 0)),
        out_shape=jax.ShapeDtypeStruct((_H, _H, c), jnp.float32),
    )(xe, me)


def _taps(w):
    return jnp.transpose(w, (2, 3, 1, 0)).reshape(9, w.shape[1], w.shape[0])


def kernel(batch, pooling_mask, w1, b1, w2, b2, w3, b3, w4, b4, w5, b5,
           w6, b6, w7, b7, w8, b8, w9, b9, w10, b10):
    mask3 = pooling_mask[0].astype(jnp.float32)[:, :, None]

    x0p = jnp.pad(batch[0], ((0, 0), (1, 1), (1, 1)))
    x1pl = pl.pallas_call(
        _conv1_kernel,
        in_specs=[pl.BlockSpec(memory_space=pltpu.VMEM),
                  pl.BlockSpec(memory_space=pltpu.SMEM),
                  pl.BlockSpec(memory_space=pltpu.SMEM)],
        out_specs=pl.BlockSpec(memory_space=pltpu.VMEM),
        out_shape=jax.ShapeDtypeStruct((64, 224, 224), jnp.float32),
    )(x0p, w1.reshape(64, 27), b1)
    x1 = jnp.transpose(x1pl, (1, 2, 0))

    x2f = _dconv(_padded(x1, 1), _taps(w2), b2.reshape(1, -1), rows=16,
                 out_h=224, out_w=224, d=1)
    x2 = _maxpool(x2f, rows=16)
    x3 = _dconv(_padded(x2, 1), _taps(w3), b3.reshape(1, -1), rows=16,
                out_h=_H, out_w=_H, d=1)
    x4 = _dconv(_padded(x3, 1), _taps(w4), b4.reshape(1, -1), rows=16,
                out_h=_H, out_w=_H, d=1)

    n1 = _pool_mask(x4, mask3, rows=16, d=1)
    g5 = _gconv(n1, w5, b5.reshape(1, -1), mask3, rows=16, d=2)
    g6 = _gconv(g5, w6, b6.reshape(1, -1), mask3, rows=16, d=2)
    g7 = _gconv(g6, w7, b7.reshape(1, -1), mask3, rows=16, d=2)
    n2 = _pool_mask(g7, mask3, rows=16, d=2)
    g8 = _gconv(n2, w8, b8.reshape(1, -1), mask3, rows=16, d=4)
    g9 = _gconv(g8, w9, b9.reshape(1, -1), mask3, rows=16, d=4)
    g10 = _gconv(g9, w10, b10.reshape(1, -1), mask3, rows=16, d=4)

    return jnp.transpose(g10, (2, 0, 1))[None]
